# baseline (speedup 1.0000x reference)
import functools
import jax
import jax.numpy as jnp
from jax import lax
from jax.experimental import pallas as pl
from jax.experimental.pallas import tpu as pltpu, tpu_sc as plsc

N = 10000
NP = 10240
E = 160000
H = 128
H2 = 2 * H

NC = 2
NS = 16
NW = NC * NS
CHUNK = 64
NCHUNKP = 2560
EP = NCHUNKP * CHUNK
CPW = NCHUNKP // NW
ROWS_PER_SUB = NP // NS

BN = 1000
BE = 2048

_mesh = plsc.VectorSubcoreMesh(core_axis_name="c", subcore_axis_name="s")


def _sc_gather_body(a_hbm, b_hbm, dsti, srci, g_hbm,
                    idxa_d, idxa_s, ba0, bb0, ba1, bb1,
                    sem_a, sem_b, sem_w):
    c = lax.axis_index("c")
    s = lax.axis_index("s")
    wid = s * NC + c
    bas = [ba0, ba1]
    bbs = [bb0, bb1]

    pltpu.sync_copy(dsti.at[wid], idxa_d)
    pltpu.sync_copy(srci.at[wid], idxa_s)

    def start_gather(j, b):
        pltpu.make_async_copy(a_hbm.at[idxa_d.at[j]], bas[b], sem_a).start()
        pltpu.make_async_copy(b_hbm.at[idxa_s.at[j]], bbs[b], sem_b).start()

    def wait_gather(j, b):
        pltpu.make_async_copy(a_hbm.at[idxa_d.at[j]], bas[b], sem_a).wait()
        pltpu.make_async_copy(b_hbm.at[idxa_s.at[j]], bbs[b], sem_b).wait()

    start_gather(0, 0)

    def pair_body(jj, carry):
        for bpar in range(2):
            j = jj * 2 + bpar
            cur, nxt = bpar, 1 - bpar

            wait_gather(j, cur)

            @pl.when(j >= 1)
            def _():
                pltpu.make_async_copy(
                    bas[nxt], g_hbm.at[pl.ds(0, CHUNK)], sem_w).wait()

            @pl.when(j + 1 < CPW)
            def _():
                start_gather(j + 1, nxt)

            def row_body(i, carry2):
                for g in range(H2 // 16):
                    sl = pl.ds(g * 16, 16)
                    bas[cur][i, sl] = bas[cur][i, sl] + bbs[cur][i, sl]
                return carry2

            lax.fori_loop(0, CHUNK, row_body, 0)

            rows = pl.ds((wid * CPW + j) * CHUNK, CHUNK)
            pltpu.make_async_copy(bas[cur], g_hbm.at[rows], sem_w).start()
        return carry

    lax.fori_loop(0, CPW // 2, pair_body, 0)
    pltpu.make_async_copy(bas[1], g_hbm.at[pl.ds(0, CHUNK)], sem_w).wait()


_sc_gather = pl.kernel(
    _sc_gather_body,
    out_type=jax.ShapeDtypeStruct((EP, H2), jnp.float32),
    mesh=_mesh,
    scratch_types=[
        pltpu.VMEM((CPW, CHUNK), jnp.int32),
        pltpu.VMEM((CPW, CHUNK), jnp.int32),
        pltpu.VMEM((CHUNK, H2), jnp.float32),
        pltpu.VMEM((CHUNK, H2), jnp.float32),
        pltpu.VMEM((CHUNK, H2), jnp.float32),
        pltpu.VMEM((CHUNK, H2), jnp.float32),
        pltpu.SemaphoreType.DMA,
        pltpu.SemaphoreType.DMA,
        pltpu.SemaphoreType.DMA,
    ],
)


def _sc_count_body(dsti, z128, cnt_out, idxa, ones_v, cnt_sh):
    c = lax.axis_index("c")
    s = lax.axis_index("s")
    wid = s * NC + c

    @pl.when(s == 0)
    def _zero():
        pltpu.sync_copy(z128, cnt_sh)

    pltpu.sync_copy(dsti.at[wid], idxa)

    def ones_body(i, carry):
        for g in range(H // 16):
            ones_v[i, pl.ds(g * 16, 16)] = jnp.full((16,), 1.0, jnp.float32)
        return carry
    lax.fori_loop(0, CHUNK, ones_body, 0)

    plsc.subcore_barrier()

    def chunk_body(j, carry):
        pltpu.sync_copy(ones_v, cnt_sh.at[idxa.at[j]], add=True)
        return carry

    lax.fori_loop(0, CPW, chunk_body, 0)
    plsc.subcore_barrier()

    rows = pl.ds(s * ROWS_PER_SUB, ROWS_PER_SUB)
    pltpu.sync_copy(cnt_sh.at[rows], cnt_out.at[c, rows])


_sc_count = pl.kernel(
    _sc_count_body,
    out_type=jax.ShapeDtypeStruct((NC, NP, H), jnp.float32),
    mesh=_mesh,
    scratch_types=[
        pltpu.VMEM((CPW, CHUNK), jnp.int32),
        pltpu.VMEM((CHUNK, H), jnp.float32),
        pltpu.VMEM_SHARED((NP, H), jnp.float32),
    ],
)


def _sc_scatter_body(m2_hbm, dsti, z128, agg_out,
                     idxa, mb0, mb1, agg_sh, sem_m):
    c = lax.axis_index("c")
    s = lax.axis_index("s")
    wid = s * NC + c
    mbs = [mb0, mb1]

    @pl.when(s == 0)
    def _zero():
        pltpu.sync_copy(z128, agg_sh)

    pltpu.sync_copy(dsti.at[wid], idxa)

    def row_of(j):
        return (wid * CPW + j) * CHUNK

    pltpu.make_async_copy(
        m2_hbm.at[pl.ds(row_of(0), CHUNK)], mb0, sem_m).start()

    plsc.subcore_barrier()

    def pair_body(jj, carry):
        for bpar in range(2):
            j = jj * 2 + bpar
            cur, nxt = bpar, 1 - bpar

            pltpu.make_async_copy(
                m2_hbm.at[pl.ds(0, CHUNK)], mbs[cur], sem_m).wait()

            @pl.when(j + 1 < CPW)
            def _():
                pltpu.make_async_copy(
                    m2_hbm.at[pl.ds(row_of(j + 1), CHUNK)],
                    mbs[nxt], sem_m).start()

            pltpu.sync_copy(mbs[cur], agg_sh.at[idxa.at[j]], add=True)
        return carry

    lax.fori_loop(0, CPW // 2, pair_body, 0)
    plsc.subcore_barrier()

    rows = pl.ds(s * ROWS_PER_SUB, ROWS_PER_SUB)
    pltpu.sync_copy(agg_sh.at[rows], agg_out.at[c, rows])


_sc_scatter = pl.kernel(
    _sc_scatter_body,
    out_type=jax.ShapeDtypeStruct((NC, NP, H), jnp.float32),
    mesh=_mesh,
    scratch_types=[
        pltpu.VMEM((CPW, CHUNK), jnp.int32),
        pltpu.VMEM((CHUNK, H), jnp.float32),
        pltpu.VMEM((CHUNK, H), jnp.float32),
        pltpu.VMEM_SHARED((NP, H), jnp.float32),
        pltpu.SemaphoreType.DMA,
    ],
)


def _ln_relu(z, g, b):
    mu = jnp.mean(z, axis=-1, keepdims=True)
    zc = z - mu
    var = jnp.mean(zc * zc, axis=-1, keepdims=True)
    return jax.nn.relu(zc * jax.lax.rsqrt(var + 1e-5) * g + b)


def _dot(a, b):
    return jnp.dot(a, b, preferred_element_type=jnp.float32)


def _node_pre_body(feats_ref, nw_ref, nb_ref, wa_ref, wb_ref,
                   h_ref, a_ref, b_ref):
    h = _dot(feats_ref[...], nw_ref[...]) + nb_ref[...]
    h_ref[...] = h
    a_ref[...] = _dot(h, wa_ref[...])
    b_ref[...] = _dot(h, wb_ref[...])


def _node_pre(feats, nw, nb, wa, wb):
    full = lambda shape: pl.BlockSpec(shape, lambda i: (0,) * len(shape))
    return pl.pallas_call(
        _node_pre_body,
        grid=(N // BN,),
        in_specs=[
            pl.BlockSpec((BN, H), lambda i: (i, 0)),
            full((H, H)), full((1, H)), full((H, H2)), full((H, H2)),
        ],
        out_specs=[
            pl.BlockSpec((BN, H), lambda i: (i, 0)),
            pl.BlockSpec((BN, H2), lambda i: (i, 0)),
            pl.BlockSpec((BN, H2), lambda i: (i, 0)),
        ],
        out_shape=[
            jax.ShapeDtypeStruct((N, H), jnp.float32),
            jax.ShapeDtypeStruct((NP, H2), jnp.float32),
            jax.ShapeDtypeStruct((NP, H2), jnp.float32),
        ],
    )(feats, nw, nb, wa, wb)


def _edge_mlp_body(g_ref, ea_ref, c_ref, cb_ref, w2_ref, b2_ref,
                   g1_ref, be1_ref, g2_ref, be2_ref, out_ref):
    z = g_ref[...] + _dot(ea_ref[...], c_ref[...]) + cb_ref[...]
    m = _ln_relu(z, g1_ref[...], be1_ref[...])
    m2 = _dot(m, w2_ref[...]) + b2_ref[...]
    out_ref[...] = _ln_relu(m2, g2_ref[...], be2_ref[...])


def _edge_mlp(g, ea, cmat, cbias, w2, b2, g1, be1, g2, be2):
    full = lambda shape: pl.BlockSpec(shape, lambda i: (0,) * len(shape))
    return pl.pallas_call(
        _edge_mlp_body,
        grid=(EP // BE,),
        in_specs=[
            pl.BlockSpec((BE, H2), lambda i: (i, 0)),
            pl.BlockSpec((BE, 16), lambda i: (i, 0)),
            full((16, H2)), full((1, H2)), full((H2, H)), full((1, H)),
            full((1, H2)), full((1, H2)), full((1, H)), full((1, H)),
        ],
        out_specs=pl.BlockSpec((BE, H), lambda i: (i, 0)),
        out_shape=jax.ShapeDtypeStruct((EP, H), jnp.float32),
    )(g, ea, cmat, cbias, w2, b2, g1, be1, g2, be2)


def _make_post_body(with_ab):
    def body(*refs):
        if with_ab:
            (aggp_ref, cntp_ref, h_ref, pw_ref, pb_ref, ng_ref, nb_ref,
             wa_ref, wb_ref, out_ref, a_ref, b_ref) = refs
        else:
            (aggp_ref, cntp_ref, h_ref, pw_ref, pb_ref, ng_ref, nb_ref,
             out_ref) = refs
        agg = aggp_ref[0] + aggp_ref[1]
        cnt = cntp_ref[0, :, 0] + cntp_ref[1, :, 0]
        agg = agg / jnp.maximum(cnt, 1.0)[:, None]
        o = _dot(agg, pw_ref[...]) + pb_ref[...]
        hn = _ln_relu(o, ng_ref[...], nb_ref[...]) + h_ref[...]
        out_ref[...] = hn
        if with_ab:
            a_ref[...] = _dot(hn, wa_ref[...])
            b_ref[...] = _dot(hn, wb_ref[...])
    return body


def _post(aggp, cntp, h, pw, pb, ng, nb, wa=None, wb=None):
    with_ab = wa is not None
    full = lambda shape: pl.BlockSpec(shape, lambda i: (0,) * len(shape))
    in_specs = [
        pl.BlockSpec((NC, BN, H), lambda i: (0, i, 0)),
        pl.BlockSpec((NC, BN, H), lambda i: (0, i, 0)),
        pl.BlockSpec((BN, H), lambda i: (i, 0)),
        full((H, H)), full((1, H)), full((1, H)), full((1, H)),
    ]
    out_specs = [pl.BlockSpec((BN, H), lambda i: (i, 0))]
    out_shape = [jax.ShapeDtypeStruct((N, H), jnp.float32)]
    args = [aggp, cntp, h, pw, pb, ng, nb]
    if with_ab:
        in_specs += [full((H, H2)), full((H, H2))]
        out_specs += [pl.BlockSpec((BN, H2), lambda i: (i, 0)),
                      pl.BlockSpec((BN, H2), lambda i: (i, 0))]
        out_shape += [jax.ShapeDtypeStruct((NP, H2), jnp.float32),
                      jax.ShapeDtypeStruct((NP, H2), jnp.float32)]
        args += [wa, wb]
    out = pl.pallas_call(
        _make_post_body(with_ab),
        grid=(N // BN,),
        in_specs=in_specs,
        out_specs=out_specs,
        out_shape=out_shape,
    )(*args)
    return out


def kernel(x, pos, edge_attr, params, edge_index, batch):
    feats = jnp.concatenate([x, pos], axis=1)
    src = edge_index[0]
    dst = edge_index[1]
    pad = jnp.full((EP - E,), NP - 1, jnp.int32)
    dsti = jnp.concatenate([dst, pad]).reshape(NW, CPW, CHUNK)
    srci = jnp.concatenate([src, pad]).reshape(NW, CPW, CHUNK)
    eap = jnp.pad(edge_attr, ((0, EP - E), (0, 0)))

    row = lambda v: v.reshape(1, -1)
    wa, wb, cmat, cbias = [], [], [], []
    for lp in params['layers']:
        w1 = lp['W1']
        w1a, w1b, w1c = w1[:H], w1[H:2 * H], w1[2 * H:]
        wa.append(w1a - w1b)
        wb.append(w1b)
        cmat.append(params['edge_W'] @ w1c)
        cbias.append(row(params['edge_b'] @ w1c + lp['b1']))

    h, a, b = _node_pre(feats, params['node_W'], row(params['node_b']),
                        wa[0], wb[0])

    z128 = jnp.zeros((NP, H), jnp.float32)
    cntp = _sc_count(dsti, z128)

    for li, lp in enumerate(params['layers']):
        g = _sc_gather(a, b, dsti, srci)
        m2 = _edge_mlp(g, eap, cmat[li], cbias[li], lp['W2'],
                       row(lp['b2']), row(lp['g1']), row(lp['be1']),
                       row(lp['g2']), row(lp['be2']))
        aggp = _sc_scatter(m2, dsti, z128)
        if li == 0:
            h, a, b = _post(aggp, cntp, h, lp['pW'], row(lp['pb']),
                            row(lp['ng']), row(lp['nb']),
                            wa[1], wb[1])
        else:
            h = _post(aggp, cntp, h, lp['pW'], row(lp['pb']),
                      row(lp['ng']), row(lp['nb']))[0]
    return h


# speedup vs baseline: 1.0946x; 1.0946x over previous
import functools
import jax
import jax.numpy as jnp
from jax import lax
from jax.experimental import pallas as pl
from jax.experimental.pallas import tpu as pltpu, tpu_sc as plsc

N = 10000
NP = 10240
E = 160000
H = 128
H2 = 2 * H

NC = 2
NS = 16
NW = NC * NS
CHUNK = 128
NCHUNKP = 1280
EP = NCHUNKP * CHUNK
CPW = NCHUNKP // NW
ROWS_PER_SUB = NP // NS

BN = 1000
BE = 2048

_mesh = plsc.VectorSubcoreMesh(core_axis_name="c", subcore_axis_name="s")


def _sc_gather_body(a_hbm, b_hbm, dsti, srci, g_hbm,
                    idx_d, idx_s, buf_a, buf_b, sem_a, sem_b):
    c = lax.axis_index("c")
    s = lax.axis_index("s")
    wid = s * NC + c

    def chunk_body(j, carry):
        ch = j * NW + wid
        pltpu.sync_copy(dsti.at[ch, 0], idx_d)
        pltpu.sync_copy(srci.at[ch, 0], idx_s)
        cp_a = pltpu.async_copy(a_hbm.at[idx_d], buf_a, sem_a)
        cp_b = pltpu.async_copy(b_hbm.at[idx_s], buf_b, sem_b)
        cp_a.wait()
        cp_b.wait()

        def row_body(i, carry2):
            for g in range(H2 // 16):
                sl = pl.ds(g * 16, 16)
                buf_a[i, sl] = buf_a[i, sl] + buf_b[i, sl]
            return carry2

        lax.fori_loop(0, CHUNK, row_body, 0)
        pltpu.sync_copy(buf_a, g_hbm.at[pl.ds(ch * CHUNK, CHUNK)])
        return carry

    lax.fori_loop(0, CPW, chunk_body, 0)


_sc_gather = pl.kernel(
    _sc_gather_body,
    out_type=jax.ShapeDtypeStruct((EP, H2), jnp.float32),
    mesh=_mesh,
    scratch_types=[
        pltpu.VMEM((CHUNK,), jnp.int32),
        pltpu.VMEM((CHUNK,), jnp.int32),
        pltpu.VMEM((CHUNK, H2), jnp.float32),
        pltpu.VMEM((CHUNK, H2), jnp.float32),
        pltpu.SemaphoreType.DMA,
        pltpu.SemaphoreType.DMA,
    ],
)


def _sc_count_body(dsti, z128, cnt_out, idx, ones_v, cnt_sh):
    c = lax.axis_index("c")
    s = lax.axis_index("s")
    wid = s * NC + c

    @pl.when(s == 0)
    def _zero():
        pltpu.sync_copy(z128, cnt_sh)

    def ones_body(i, carry):
        for g in range(H // 16):
            ones_v[i, pl.ds(g * 16, 16)] = jnp.full((16,), 1.0, jnp.float32)
        return carry
    lax.fori_loop(0, CHUNK, ones_body, 0)

    plsc.subcore_barrier()

    def chunk_body(j, carry):
        ch = j * NW + wid
        pltpu.sync_copy(dsti.at[ch, 0], idx)
        pltpu.sync_copy(ones_v, cnt_sh.at[idx], add=True)
        return carry

    lax.fori_loop(0, CPW, chunk_body, 0)
    plsc.subcore_barrier()

    rows = pl.ds(s * ROWS_PER_SUB, ROWS_PER_SUB)
    pltpu.sync_copy(cnt_sh.at[rows], cnt_out.at[c, rows])


_sc_count = pl.kernel(
    _sc_count_body,
    out_type=jax.ShapeDtypeStruct((NC, NP, H), jnp.float32),
    mesh=_mesh,
    scratch_types=[
        pltpu.VMEM((CHUNK,), jnp.int32),
        pltpu.VMEM((CHUNK, H), jnp.float32),
        pltpu.VMEM_SHARED((NP, H), jnp.float32),
    ],
)


def _sc_scatter_body(m2_hbm, dsti, z128, agg_out,
                     idx0, idx1, mb0, mb1, agg_sh, sem_m):
    c = lax.axis_index("c")
    s = lax.axis_index("s")
    wid = s * NC + c
    idxs = [idx0, idx1]
    mbs = [mb0, mb1]

    @pl.when(s == 0)
    def _zero():
        pltpu.sync_copy(z128, agg_sh)

    def chunk_of(j):
        return j * NW + wid

    pltpu.sync_copy(dsti.at[chunk_of(0), 0], idx0)
    pltpu.make_async_copy(
        m2_hbm.at[pl.ds(chunk_of(0) * CHUNK, CHUNK)], mb0, sem_m).start()
    pltpu.sync_copy(dsti.at[chunk_of(1), 0], idx1)

    plsc.subcore_barrier()

    def pair_body(jj, carry):
        for bpar in range(2):
            j = jj * 2 + bpar
            cur, nxt = bpar, 1 - bpar

            pltpu.make_async_copy(
                m2_hbm.at[pl.ds(0, CHUNK)], mbs[cur], sem_m).wait()

            @pl.when(j + 1 < CPW)
            def _():
                pltpu.make_async_copy(
                    m2_hbm.at[pl.ds(chunk_of(j + 1) * CHUNK, CHUNK)],
                    mbs[nxt], sem_m).start()

            pltpu.sync_copy(mbs[cur], agg_sh.at[idxs[cur]], add=True)

            @pl.when(j + 2 < CPW)
            def _():
                pltpu.sync_copy(dsti.at[chunk_of(j + 2), 0], idxs[cur])
        return carry

    lax.fori_loop(0, CPW // 2, pair_body, 0)
    plsc.subcore_barrier()

    rows = pl.ds(s * ROWS_PER_SUB, ROWS_PER_SUB)
    pltpu.sync_copy(agg_sh.at[rows], agg_out.at[c, rows])


_sc_scatter = pl.kernel(
    _sc_scatter_body,
    out_type=jax.ShapeDtypeStruct((NC, NP, H), jnp.float32),
    mesh=_mesh,
    scratch_types=[
        pltpu.VMEM((CHUNK,), jnp.int32),
        pltpu.VMEM((CHUNK,), jnp.int32),
        pltpu.VMEM((CHUNK, H), jnp.float32),
        pltpu.VMEM((CHUNK, H), jnp.float32),
        pltpu.VMEM_SHARED((NP, H), jnp.float32),
        pltpu.SemaphoreType.DMA,
    ],
)


def _ln_relu(z, g, b):
    mu = jnp.mean(z, axis=-1, keepdims=True)
    zc = z - mu
    var = jnp.mean(zc * zc, axis=-1, keepdims=True)
    return jax.nn.relu(zc * jax.lax.rsqrt(var + 1e-5) * g + b)


def _dot(a, b):
    return jnp.dot(a, b, preferred_element_type=jnp.float32)


def _node_pre_body(feats_ref, nw_ref, nb_ref, wa_ref, wb_ref,
                   h_ref, a_ref, b_ref):
    h = _dot(feats_ref[...], nw_ref[...]) + nb_ref[...]
    h_ref[...] = h
    a_ref[...] = _dot(h, wa_ref[...])
    b_ref[...] = _dot(h, wb_ref[...])


def _node_pre(feats, nw, nb, wa, wb):
    full = lambda shape: pl.BlockSpec(shape, lambda i: (0,) * len(shape))
    return pl.pallas_call(
        _node_pre_body,
        grid=(N // BN,),
        in_specs=[
            pl.BlockSpec((BN, H), lambda i: (i, 0)),
            full((H, H)), full((1, H)), full((H, H2)), full((H, H2)),
        ],
        out_specs=[
            pl.BlockSpec((BN, H), lambda i: (i, 0)),
            pl.BlockSpec((BN, H2), lambda i: (i, 0)),
            pl.BlockSpec((BN, H2), lambda i: (i, 0)),
        ],
        out_shape=[
            jax.ShapeDtypeStruct((N, H), jnp.float32),
            jax.ShapeDtypeStruct((NP, H2), jnp.float32),
            jax.ShapeDtypeStruct((NP, H2), jnp.float32),
        ],
    )(feats, nw, nb, wa, wb)


def _edge_mlp_body(g_ref, ea_ref, c_ref, cb_ref, w2_ref, b2_ref,
                   g1_ref, be1_ref, g2_ref, be2_ref, out_ref):
    z = g_ref[...] + _dot(ea_ref[...], c_ref[...]) + cb_ref[...]
    m = _ln_relu(z, g1_ref[...], be1_ref[...])
    m2 = _dot(m, w2_ref[...]) + b2_ref[...]
    out_ref[...] = _ln_relu(m2, g2_ref[...], be2_ref[...])


def _edge_mlp(g, ea, cmat, cbias, w2, b2, g1, be1, g2, be2):
    full = lambda shape: pl.BlockSpec(shape, lambda i: (0,) * len(shape))
    return pl.pallas_call(
        _edge_mlp_body,
        grid=(EP // BE,),
        in_specs=[
            pl.BlockSpec((BE, H2), lambda i: (i, 0)),
            pl.BlockSpec((BE, 16), lambda i: (i, 0)),
            full((16, H2)), full((1, H2)), full((H2, H)), full((1, H)),
            full((1, H2)), full((1, H2)), full((1, H)), full((1, H)),
        ],
        out_specs=pl.BlockSpec((BE, H), lambda i: (i, 0)),
        out_shape=jax.ShapeDtypeStruct((EP, H), jnp.float32),
    )(g, ea, cmat, cbias, w2, b2, g1, be1, g2, be2)


def _make_post_body(with_ab):
    def body(*refs):
        if with_ab:
            (aggp_ref, cntp_ref, h_ref, pw_ref, pb_ref, ng_ref, nb_ref,
             wa_ref, wb_ref, out_ref, a_ref, b_ref) = refs
        else:
            (aggp_ref, cntp_ref, h_ref, pw_ref, pb_ref, ng_ref, nb_ref,
             out_ref) = refs
        agg = aggp_ref[0] + aggp_ref[1]
        cnt = cntp_ref[0, :, 0] + cntp_ref[1, :, 0]
        agg = agg / jnp.maximum(cnt, 1.0)[:, None]
        o = _dot(agg, pw_ref[...]) + pb_ref[...]
        hn = _ln_relu(o, ng_ref[...], nb_ref[...]) + h_ref[...]
        out_ref[...] = hn
        if with_ab:
            a_ref[...] = _dot(hn, wa_ref[...])
            b_ref[...] = _dot(hn, wb_ref[...])
    return body


def _post(aggp, cntp, h, pw, pb, ng, nb, wa=None, wb=None):
    with_ab = wa is not None
    full = lambda shape: pl.BlockSpec(shape, lambda i: (0,) * len(shape))
    in_specs = [
        pl.BlockSpec((NC, BN, H), lambda i: (0, i, 0)),
        pl.BlockSpec((NC, BN, H), lambda i: (0, i, 0)),
        pl.BlockSpec((BN, H), lambda i: (i, 0)),
        full((H, H)), full((1, H)), full((1, H)), full((1, H)),
    ]
    out_specs = [pl.BlockSpec((BN, H), lambda i: (i, 0))]
    out_shape = [jax.ShapeDtypeStruct((N, H), jnp.float32)]
    args = [aggp, cntp, h, pw, pb, ng, nb]
    if with_ab:
        in_specs += [full((H, H2)), full((H, H2))]
        out_specs += [pl.BlockSpec((BN, H2), lambda i: (i, 0)),
                      pl.BlockSpec((BN, H2), lambda i: (i, 0))]
        out_shape += [jax.ShapeDtypeStruct((NP, H2), jnp.float32),
                      jax.ShapeDtypeStruct((NP, H2), jnp.float32)]
        args += [wa, wb]
    out = pl.pallas_call(
        _make_post_body(with_ab),
        grid=(N // BN,),
        in_specs=in_specs,
        out_specs=out_specs,
        out_shape=out_shape,
    )(*args)
    return out


def kernel(x, pos, edge_attr, params, edge_index, batch):
    feats = jnp.concatenate([x, pos], axis=1)
    src = edge_index[0]
    dst = edge_index[1]
    pad = jnp.full((EP - E,), NP - 1, jnp.int32)
    dsti = jnp.concatenate([dst, pad]).reshape(NCHUNKP, 1, CHUNK)
    srci = jnp.concatenate([src, pad]).reshape(NCHUNKP, 1, CHUNK)
    eap = jnp.pad(edge_attr, ((0, EP - E), (0, 0)))

    row = lambda v: v.reshape(1, -1)
    wa, wb, cmat, cbias = [], [], [], []
    for lp in params['layers']:
        w1 = lp['W1']
        w1a, w1b, w1c = w1[:H], w1[H:2 * H], w1[2 * H:]
        wa.append(w1a - w1b)
        wb.append(w1b)
        cmat.append(params['edge_W'] @ w1c)
        cbias.append(row(params['edge_b'] @ w1c + lp['b1']))

    h, a, b = _node_pre(feats, params['node_W'], row(params['node_b']),
                        wa[0], wb[0])

    z128 = jnp.zeros((NP, H), jnp.float32)
    cntp = _sc_count(dsti, z128)

    for li, lp in enumerate(params['layers']):
        g = _sc_gather(a, b, dsti, srci)
        m2 = _edge_mlp(g, eap, cmat[li], cbias[li], lp['W2'],
                       row(lp['b2']), row(lp['g1']), row(lp['be1']),
                       row(lp['g2']), row(lp['be2']))
        aggp = _sc_scatter(m2, dsti, z128)
        if li == 0:
            h, a, b = _post(aggp, cntp, h, lp['pW'], row(lp['pb']),
                            row(lp['ng']), row(lp['nb']),
                            wa[1], wb[1])
        else:
            h = _post(aggp, cntp, h, lp['pW'], row(lp['pb']),
                      row(lp['ng']), row(lp['nb']))[0]
    return h


# speedup vs baseline: 1.3467x; 1.2303x over previous
import functools
import jax
import jax.numpy as jnp
from jax import lax
from jax.experimental import pallas as pl
from jax.experimental.pallas import tpu as pltpu, tpu_sc as plsc

N = 10000
NP = 10240
E = 160000
H = 128
H2 = 2 * H

NC = 2
NS = 16
NW = NC * NS
CHUNK = 128
NCHUNKP = 1280
EP = NCHUNKP * CHUNK
CPW = NCHUNKP // NW
ROWS_PER_SUB = NP // NS

BN = 1000
BE = 2048

_mesh = plsc.VectorSubcoreMesh(core_axis_name="c", subcore_axis_name="s")


def _sc_gather_body(a_hbm, b_hbm, dsti, srci, g_hbm,
                    idx_d, idx_s, buf_a, buf_b, sem_a, sem_b):
    c = lax.axis_index("c")
    s = lax.axis_index("s")
    wid = s * NC + c

    def chunk_body(j, carry):
        ch = j * NW + wid
        pltpu.sync_copy(dsti.at[ch, 0], idx_d)
        pltpu.sync_copy(srci.at[ch, 0], idx_s)
        cp_a = pltpu.async_copy(a_hbm.at[idx_d], buf_a, sem_a)
        cp_b = pltpu.async_copy(b_hbm.at[idx_s], buf_b, sem_b)
        cp_a.wait()
        cp_b.wait()

        def row_body(i, carry2):
            for g in range(H2 // 16):
                sl = pl.ds(g * 16, 16)
                buf_a[i, sl] = buf_a[i, sl] + buf_b[i, sl]
            return carry2

        lax.fori_loop(0, CHUNK, row_body, 0)
        pltpu.sync_copy(buf_a, g_hbm.at[pl.ds(ch * CHUNK, CHUNK)])
        return carry

    lax.fori_loop(0, CPW, chunk_body, 0)


_sc_gather = pl.kernel(
    _sc_gather_body,
    out_type=jax.ShapeDtypeStruct((EP, H2), jnp.float32),
    mesh=_mesh,
    scratch_types=[
        pltpu.VMEM((CHUNK,), jnp.int32),
        pltpu.VMEM((CHUNK,), jnp.int32),
        pltpu.VMEM((CHUNK, H2), jnp.float32),
        pltpu.VMEM((CHUNK, H2), jnp.float32),
        pltpu.SemaphoreType.DMA,
        pltpu.SemaphoreType.DMA,
    ],
)


def _sc_count_body(dsti, z128, cnt_out, idx, ones_v, cnt_sh):
    c = lax.axis_index("c")
    s = lax.axis_index("s")
    wid = s * NC + c

    @pl.when(s == 0)
    def _zero():
        pltpu.sync_copy(z128, cnt_sh)

    def ones_body(i, carry):
        for g in range(H // 16):
            ones_v[i, pl.ds(g * 16, 16)] = jnp.full((16,), 1.0, jnp.float32)
        return carry
    lax.fori_loop(0, CHUNK, ones_body, 0)

    plsc.subcore_barrier()

    def chunk_body(j, carry):
        ch = j * NW + wid
        pltpu.sync_copy(dsti.at[ch, 0], idx)
        pltpu.sync_copy(ones_v, cnt_sh.at[idx], add=True)
        return carry

    lax.fori_loop(0, CPW, chunk_body, 0)
    plsc.subcore_barrier()

    rows = pl.ds(s * ROWS_PER_SUB, ROWS_PER_SUB)
    pltpu.sync_copy(cnt_sh.at[rows], cnt_out.at[c, rows])


_sc_count = pl.kernel(
    _sc_count_body,
    out_type=jax.ShapeDtypeStruct((NC, NP, H), jnp.float32),
    mesh=_mesh,
    scratch_types=[
        pltpu.VMEM((CHUNK,), jnp.int32),
        pltpu.VMEM((CHUNK, H), jnp.float32),
        pltpu.VMEM_SHARED((NP, H), jnp.float32),
    ],
)


def _sc_scatter_body(m2_hbm, dsti, z128, agg_out,
                     idx0, idx1, mb0, mb1, agg_sh, sem_m):
    c = lax.axis_index("c")
    s = lax.axis_index("s")
    wid = s * NC + c
    idxs = [idx0, idx1]
    mbs = [mb0, mb1]

    @pl.when(s == 0)
    def _zero():
        pltpu.sync_copy(z128, agg_sh)

    def chunk_of(j):
        return j * NW + wid

    pltpu.sync_copy(dsti.at[chunk_of(0), 0], idx0)
    pltpu.make_async_copy(
        m2_hbm.at[pl.ds(chunk_of(0) * CHUNK, CHUNK)], mb0, sem_m).start()
    pltpu.sync_copy(dsti.at[chunk_of(1), 0], idx1)

    plsc.subcore_barrier()

    def pair_body(jj, carry):
        for bpar in range(2):
            j = jj * 2 + bpar
            cur, nxt = bpar, 1 - bpar

            pltpu.make_async_copy(
                m2_hbm.at[pl.ds(0, CHUNK)], mbs[cur], sem_m).wait()

            @pl.when(j + 1 < CPW)
            def _():
                pltpu.make_async_copy(
                    m2_hbm.at[pl.ds(chunk_of(j + 1) * CHUNK, CHUNK)],
                    mbs[nxt], sem_m).start()

            pltpu.sync_copy(mbs[cur], agg_sh.at[idxs[cur]], add=True)

            @pl.when(j + 2 < CPW)
            def _():
                pltpu.sync_copy(dsti.at[chunk_of(j + 2), 0], idxs[cur])
        return carry

    lax.fori_loop(0, CPW // 2, pair_body, 0)
    plsc.subcore_barrier()

    rows = pl.ds(s * ROWS_PER_SUB, ROWS_PER_SUB)
    pltpu.sync_copy(agg_sh.at[rows], agg_out.at[c, rows])


_sc_scatter = pl.kernel(
    _sc_scatter_body,
    out_type=jax.ShapeDtypeStruct((NC, NP, H), jnp.float32),
    mesh=_mesh,
    scratch_types=[
        pltpu.VMEM((CHUNK,), jnp.int32),
        pltpu.VMEM((CHUNK,), jnp.int32),
        pltpu.VMEM((CHUNK, H), jnp.float32),
        pltpu.VMEM((CHUNK, H), jnp.float32),
        pltpu.VMEM_SHARED((NP, H), jnp.float32),
        pltpu.SemaphoreType.DMA,
    ],
)


def _ln_relu(z, g, b):
    mu = jnp.mean(z, axis=-1, keepdims=True)
    zc = z - mu
    var = jnp.mean(zc * zc, axis=-1, keepdims=True)
    return jax.nn.relu(zc * jax.lax.rsqrt(var + 1e-5) * g + b)


def _dot(a, b):
    return jnp.dot(a, b, preferred_element_type=jnp.float32)


def _node_pre_body(feats_ref, nw_ref, nb_ref, wa_ref, wb_ref,
                   h_ref, a_ref, b_ref):
    h = _dot(feats_ref[...], nw_ref[...]) + nb_ref[...]
    h_ref[...] = h
    a_ref[...] = _dot(h, wa_ref[...])
    b_ref[...] = _dot(h, wb_ref[...])


def _node_pre(feats, nw, nb, wa, wb):
    full = lambda shape: pl.BlockSpec(shape, lambda i: (0,) * len(shape))
    return pl.pallas_call(
        _node_pre_body,
        grid=(N // BN,),
        in_specs=[
            pl.BlockSpec((BN, H), lambda i: (i, 0)),
            full((H, H)), full((1, H)), full((H, H2)), full((H, H2)),
        ],
        out_specs=[
            pl.BlockSpec((BN, H), lambda i: (i, 0)),
            pl.BlockSpec((BN, H2), lambda i: (i, 0)),
            pl.BlockSpec((BN, H2), lambda i: (i, 0)),
        ],
        out_shape=[
            jax.ShapeDtypeStruct((N, H), jnp.float32),
            jax.ShapeDtypeStruct((NP, H2), jnp.float32),
            jax.ShapeDtypeStruct((NP, H2), jnp.float32),
        ],
    )(feats, nw, nb, wa, wb)


def _edge_mlp_body(g_ref, ea_ref, c_ref, cb_ref, w2_ref, b2_ref,
                   g1_ref, be1_ref, g2_ref, be2_ref, out_ref):
    z = g_ref[...] + _dot(ea_ref[...], c_ref[...]) + cb_ref[...]
    m = _ln_relu(z, g1_ref[...], be1_ref[...])
    m2 = _dot(m, w2_ref[...]) + b2_ref[...]
    out_ref[...] = _ln_relu(m2, g2_ref[...], be2_ref[...])


def _edge_mlp(g, ea, cmat, cbias, w2, b2, g1, be1, g2, be2):
    full = lambda shape: pl.BlockSpec(shape, lambda i: (0,) * len(shape))
    return pl.pallas_call(
        _edge_mlp_body,
        grid=(EP // BE,),
        in_specs=[
            pl.BlockSpec((BE, H2), lambda i: (i, 0)),
            pl.BlockSpec((BE, 16), lambda i: (i, 0)),
            full((16, H2)), full((1, H2)), full((H2, H)), full((1, H)),
            full((1, H2)), full((1, H2)), full((1, H)), full((1, H)),
        ],
        out_specs=pl.BlockSpec((BE, H), lambda i: (i, 0)),
        out_shape=jax.ShapeDtypeStruct((EP, H), jnp.float32),
    )(g, ea, cmat, cbias, w2, b2, g1, be1, g2, be2)


def _make_post_body(with_ab):
    def body(*refs):
        if with_ab:
            (aggp_ref, cntp_ref, h_ref, pw_ref, pb_ref, ng_ref, nb_ref,
             wa_ref, wb_ref, out_ref, a_ref, b_ref) = refs
        else:
            (aggp_ref, cntp_ref, h_ref, pw_ref, pb_ref, ng_ref, nb_ref,
             out_ref) = refs
        agg = aggp_ref[0] + aggp_ref[1]
        cnt = cntp_ref[0, :, 0] + cntp_ref[1, :, 0]
        agg = agg / jnp.maximum(cnt, 1.0)[:, None]
        o = _dot(agg, pw_ref[...]) + pb_ref[...]
        hn = _ln_relu(o, ng_ref[...], nb_ref[...]) + h_ref[...]
        out_ref[...] = hn
        if with_ab:
            a_ref[...] = _dot(hn, wa_ref[...])
            b_ref[...] = _dot(hn, wb_ref[...])
    return body


def _post(aggp, cntp, h, pw, pb, ng, nb, wa=None, wb=None):
    with_ab = wa is not None
    full = lambda shape: pl.BlockSpec(shape, lambda i: (0,) * len(shape))
    in_specs = [
        pl.BlockSpec((NC, BN, H), lambda i: (0, i, 0)),
        pl.BlockSpec((NC, BN, H), lambda i: (0, i, 0)),
        pl.BlockSpec((BN, H), lambda i: (i, 0)),
        full((H, H)), full((1, H)), full((1, H)), full((1, H)),
    ]
    out_specs = [pl.BlockSpec((BN, H), lambda i: (i, 0))]
    out_shape = [jax.ShapeDtypeStruct((N, H), jnp.float32)]
    args = [aggp, cntp, h, pw, pb, ng, nb]
    if with_ab:
        in_specs += [full((H, H2)), full((H, H2))]
        out_specs += [pl.BlockSpec((BN, H2), lambda i: (i, 0)),
                      pl.BlockSpec((BN, H2), lambda i: (i, 0))]
        out_shape += [jax.ShapeDtypeStruct((NP, H2), jnp.float32),
                      jax.ShapeDtypeStruct((NP, H2), jnp.float32)]
        args += [wa, wb]
    out = pl.pallas_call(
        _make_post_body(with_ab),
        grid=(N // BN,),
        in_specs=in_specs,
        out_specs=out_specs,
        out_shape=out_shape,
    )(*args)
    return out


def kernel(x, pos, edge_attr, params, edge_index, batch):
    feats = jnp.concatenate([x, pos], axis=1)
    src = edge_index[0]
    dst = edge_index[1]
    pad = N + (jnp.arange(EP - E, dtype=jnp.int32) % (NP - N))
    dsti = jnp.concatenate([dst, pad]).reshape(NCHUNKP, 1, CHUNK)
    srci = jnp.concatenate([src, pad]).reshape(NCHUNKP, 1, CHUNK)
    eap = jnp.pad(edge_attr, ((0, EP - E), (0, 0)))

    row = lambda v: v.reshape(1, -1)
    wa, wb, cmat, cbias = [], [], [], []
    for lp in params['layers']:
        w1 = lp['W1']
        w1a, w1b, w1c = w1[:H], w1[H:2 * H], w1[2 * H:]
        wa.append(w1a - w1b)
        wb.append(w1b)
        cmat.append(params['edge_W'] @ w1c)
        cbias.append(row(params['edge_b'] @ w1c + lp['b1']))

    h, a, b = _node_pre(feats, params['node_W'], row(params['node_b']),
                        wa[0], wb[0])

    z128 = jnp.zeros((NP, H), jnp.float32)
    cntp = _sc_count(dsti, z128)

    for li, lp in enumerate(params['layers']):
        g = _sc_gather(a, b, dsti, srci)
        m2 = _edge_mlp(g, eap, cmat[li], cbias[li], lp['W2'],
                       row(lp['b2']), row(lp['g1']), row(lp['be1']),
                       row(lp['g2']), row(lp['be2']))
        aggp = _sc_scatter(m2, dsti, z128)
        if li == 0:
            h, a, b = _post(aggp, cntp, h, lp['pW'], row(lp['pb']),
                            row(lp['ng']), row(lp['nb']),
                            wa[1], wb[1])
        else:
            h = _post(aggp, cntp, h, lp['pW'], row(lp['pb']),
                      row(lp['ng']), row(lp['nb']))[0]
    return h


# speedup vs baseline: 1.5261x; 1.1333x over previous
import functools
import jax
import jax.numpy as jnp
from jax import lax
from jax.experimental import pallas as pl
from jax.experimental.pallas import tpu as pltpu, tpu_sc as plsc

N = 10000
NP = 10240
E = 160000
H = 128
H2 = 2 * H

NC = 2
NS = 16
NW = NC * NS
CHUNK = 120
NCHUNKP = 1344
EP = NCHUNKP * CHUNK
CPW = NCHUNKP // NW
ROWS_PER_SUB = NP // NS

BN = 1000
BE = 2016

_mesh = plsc.VectorSubcoreMesh(core_axis_name="c", subcore_axis_name="s")


def _sc_gather_body(a_hbm, b_hbm, dsti, srci, g_hbm,
                    idx_d0, idx_s0, idx_d1, idx_s1,
                    ba0, bb0, ba1, bb1, sem_a, sem_b, sem_w):
    c = lax.axis_index("c")
    s = lax.axis_index("s")
    wid = s * NC + c
    idx_d = [idx_d0, idx_d1]
    idx_s = [idx_s0, idx_s1]
    bas = [ba0, ba1]
    bbs = [bb0, bb1]

    def chunk_of(j):
        return j * NW + wid

    pltpu.sync_copy(dsti.at[chunk_of(0), 0], idx_d0)
    pltpu.sync_copy(srci.at[chunk_of(0), 0], idx_s0)
    pltpu.make_async_copy(a_hbm.at[idx_d0], ba0, sem_a).start()
    pltpu.make_async_copy(b_hbm.at[idx_s0], bb0, sem_b).start()
    pltpu.sync_copy(dsti.at[chunk_of(1), 0], idx_d1)
    pltpu.sync_copy(srci.at[chunk_of(1), 0], idx_s1)

    def pair_body(jj, carry):
        for bpar in range(2):
            j = jj * 2 + bpar
            cur, nxt = bpar, 1 - bpar

            pltpu.make_async_copy(a_hbm.at[idx_d[cur]], bas[cur], sem_a).wait()
            pltpu.make_async_copy(b_hbm.at[idx_s[cur]], bbs[cur], sem_b).wait()

            @pl.when(j >= 1)
            def _():
                pltpu.make_async_copy(
                    bas[nxt], g_hbm.at[pl.ds(0, CHUNK)], sem_w).wait()

            @pl.when(j + 1 < CPW)
            def _():
                pltpu.make_async_copy(
                    a_hbm.at[idx_d[nxt]], bas[nxt], sem_a).start()
                pltpu.make_async_copy(
                    b_hbm.at[idx_s[nxt]], bbs[nxt], sem_b).start()

            @pl.when(j + 2 < CPW)
            def _():
                pltpu.sync_copy(dsti.at[chunk_of(j + 2), 0], idx_d[cur])
                pltpu.sync_copy(srci.at[chunk_of(j + 2), 0], idx_s[cur])

            def row_body(i, carry2):
                for g in range(H2 // 16):
                    sl = pl.ds(g * 16, 16)
                    bas[cur][i, sl] = bas[cur][i, sl] + bbs[cur][i, sl]
                return carry2

            lax.fori_loop(0, CHUNK, row_body, 0)

            pltpu.make_async_copy(
                bas[cur], g_hbm.at[pl.ds(chunk_of(j) * CHUNK, CHUNK)],
                sem_w).start()
        return carry

    lax.fori_loop(0, CPW // 2, pair_body, 0)
    pltpu.make_async_copy(bas[1], g_hbm.at[pl.ds(0, CHUNK)], sem_w).wait()


_sc_gather = pl.kernel(
    _sc_gather_body,
    out_type=jax.ShapeDtypeStruct((EP, H2), jnp.float32),
    mesh=_mesh,
    scratch_types=[
        pltpu.VMEM((CHUNK,), jnp.int32),
        pltpu.VMEM((CHUNK,), jnp.int32),
        pltpu.VMEM((CHUNK,), jnp.int32),
        pltpu.VMEM((CHUNK,), jnp.int32),
        pltpu.VMEM((CHUNK, H2), jnp.float32),
        pltpu.VMEM((CHUNK, H2), jnp.float32),
        pltpu.VMEM((CHUNK, H2), jnp.float32),
        pltpu.VMEM((CHUNK, H2), jnp.float32),
        pltpu.SemaphoreType.DMA,
        pltpu.SemaphoreType.DMA,
        pltpu.SemaphoreType.DMA,
    ],
)


def _sc_count_body(dsti, z128, cnt_out, idx, ones_v, cnt_sh):
    c = lax.axis_index("c")
    s = lax.axis_index("s")
    wid = s * NC + c

    @pl.when(s == 0)
    def _zero():
        pltpu.sync_copy(z128, cnt_sh)

    def ones_body(i, carry):
        for g in range(H // 16):
            ones_v[i, pl.ds(g * 16, 16)] = jnp.full((16,), 1.0, jnp.float32)
        return carry
    lax.fori_loop(0, CHUNK, ones_body, 0)

    plsc.subcore_barrier()

    def chunk_body(j, carry):
        ch = j * NW + wid
        pltpu.sync_copy(dsti.at[ch, 0], idx)
        pltpu.sync_copy(ones_v, cnt_sh.at[idx], add=True)
        return carry

    lax.fori_loop(0, CPW, chunk_body, 0)
    plsc.subcore_barrier()

    rows = pl.ds(s * ROWS_PER_SUB, ROWS_PER_SUB)
    pltpu.sync_copy(cnt_sh.at[rows], cnt_out.at[c, rows])


_sc_count = pl.kernel(
    _sc_count_body,
    out_type=jax.ShapeDtypeStruct((NC, NP, H), jnp.float32),
    mesh=_mesh,
    scratch_types=[
        pltpu.VMEM((CHUNK,), jnp.int32),
        pltpu.VMEM((CHUNK, H), jnp.float32),
        pltpu.VMEM_SHARED((NP, H), jnp.float32),
    ],
)


def _sc_scatter_body(m2_hbm, dsti, z128, agg_out,
                     idx0, idx1, mb0, mb1, agg_sh, sem_m):
    c = lax.axis_index("c")
    s = lax.axis_index("s")
    wid = s * NC + c
    idxs = [idx0, idx1]
    mbs = [mb0, mb1]

    @pl.when(s == 0)
    def _zero():
        pltpu.sync_copy(z128, agg_sh)

    def chunk_of(j):
        return j * NW + wid

    pltpu.sync_copy(dsti.at[chunk_of(0), 0], idx0)
    pltpu.make_async_copy(
        m2_hbm.at[pl.ds(chunk_of(0) * CHUNK, CHUNK)], mb0, sem_m).start()
    pltpu.sync_copy(dsti.at[chunk_of(1), 0], idx1)

    plsc.subcore_barrier()

    def pair_body(jj, carry):
        for bpar in range(2):
            j = jj * 2 + bpar
            cur, nxt = bpar, 1 - bpar

            pltpu.make_async_copy(
                m2_hbm.at[pl.ds(0, CHUNK)], mbs[cur], sem_m).wait()

            @pl.when(j + 1 < CPW)
            def _():
                pltpu.make_async_copy(
                    m2_hbm.at[pl.ds(chunk_of(j + 1) * CHUNK, CHUNK)],
                    mbs[nxt], sem_m).start()

            pltpu.sync_copy(mbs[cur], agg_sh.at[idxs[cur]], add=True)

            @pl.when(j + 2 < CPW)
            def _():
                pltpu.sync_copy(dsti.at[chunk_of(j + 2), 0], idxs[cur])
        return carry

    lax.fori_loop(0, CPW // 2, pair_body, 0)
    plsc.subcore_barrier()

    rows = pl.ds(s * ROWS_PER_SUB, ROWS_PER_SUB)
    pltpu.sync_copy(agg_sh.at[rows], agg_out.at[c, rows])


_sc_scatter = pl.kernel(
    _sc_scatter_body,
    out_type=jax.ShapeDtypeStruct((NC, NP, H), jnp.float32),
    mesh=_mesh,
    scratch_types=[
        pltpu.VMEM((CHUNK,), jnp.int32),
        pltpu.VMEM((CHUNK,), jnp.int32),
        pltpu.VMEM((CHUNK, H), jnp.float32),
        pltpu.VMEM((CHUNK, H), jnp.float32),
        pltpu.VMEM_SHARED((NP, H), jnp.float32),
        pltpu.SemaphoreType.DMA,
    ],
)


def _ln_relu(z, g, b):
    mu = jnp.mean(z, axis=-1, keepdims=True)
    zc = z - mu
    var = jnp.mean(zc * zc, axis=-1, keepdims=True)
    return jax.nn.relu(zc * jax.lax.rsqrt(var + 1e-5) * g + b)


def _dot(a, b):
    return jnp.dot(a, b, preferred_element_type=jnp.float32)


def _node_pre_body(feats_ref, nw_ref, nb_ref, wa_ref, wb_ref,
                   h_ref, a_ref, b_ref):
    h = _dot(feats_ref[...], nw_ref[...]) + nb_ref[...]
    h_ref[...] = h
    a_ref[...] = _dot(h, wa_ref[...])
    b_ref[...] = _dot(h, wb_ref[...])


def _node_pre(feats, nw, nb, wa, wb):
    full = lambda shape: pl.BlockSpec(shape, lambda i: (0,) * len(shape))
    return pl.pallas_call(
        _node_pre_body,
        grid=(N // BN,),
        in_specs=[
            pl.BlockSpec((BN, H), lambda i: (i, 0)),
            full((H, H)), full((1, H)), full((H, H2)), full((H, H2)),
        ],
        out_specs=[
            pl.BlockSpec((BN, H), lambda i: (i, 0)),
            pl.BlockSpec((BN, H2), lambda i: (i, 0)),
            pl.BlockSpec((BN, H2), lambda i: (i, 0)),
        ],
        out_shape=[
            jax.ShapeDtypeStruct((N, H), jnp.float32),
            jax.ShapeDtypeStruct((NP, H2), jnp.float32),
            jax.ShapeDtypeStruct((NP, H2), jnp.float32),
        ],
    )(feats, nw, nb, wa, wb)


def _edge_mlp_body(g_ref, ea_ref, c_ref, cb_ref, w2_ref, b2_ref,
                   g1_ref, be1_ref, g2_ref, be2_ref, out_ref):
    z = g_ref[...] + _dot(ea_ref[...], c_ref[...]) + cb_ref[...]
    m = _ln_relu(z, g1_ref[...], be1_ref[...])
    m2 = _dot(m, w2_ref[...]) + b2_ref[...]
    out_ref[...] = _ln_relu(m2, g2_ref[...], be2_ref[...])


def _edge_mlp(g, ea, cmat, cbias, w2, b2, g1, be1, g2, be2):
    full = lambda shape: pl.BlockSpec(shape, lambda i: (0,) * len(shape))
    return pl.pallas_call(
        _edge_mlp_body,
        grid=(EP // BE,),
        in_specs=[
            pl.BlockSpec((BE, H2), lambda i: (i, 0)),
            pl.BlockSpec((BE, 16), lambda i: (i, 0)),
            full((16, H2)), full((1, H2)), full((H2, H)), full((1, H)),
            full((1, H2)), full((1, H2)), full((1, H)), full((1, H)),
        ],
        out_specs=pl.BlockSpec((BE, H), lambda i: (i, 0)),
        out_shape=jax.ShapeDtypeStruct((EP, H), jnp.float32),
    )(g, ea, cmat, cbias, w2, b2, g1, be1, g2, be2)


def _make_post_body(with_ab):
    def body(*refs):
        if with_ab:
            (aggp_ref, cntp_ref, h_ref, pw_ref, pb_ref, ng_ref, nb_ref,
             wa_ref, wb_ref, out_ref, a_ref, b_ref) = refs
        else:
            (aggp_ref, cntp_ref, h_ref, pw_ref, pb_ref, ng_ref, nb_ref,
             out_ref) = refs
        agg = aggp_ref[0] + aggp_ref[1]
        cnt = cntp_ref[0, :, 0] + cntp_ref[1, :, 0]
        agg = agg / jnp.maximum(cnt, 1.0)[:, None]
        o = _dot(agg, pw_ref[...]) + pb_ref[...]
        hn = _ln_relu(o, ng_ref[...], nb_ref[...]) + h_ref[...]
        out_ref[...] = hn
        if with_ab:
            a_ref[...] = _dot(hn, wa_ref[...])
            b_ref[...] = _dot(hn, wb_ref[...])
    return body


def _post(aggp, cntp, h, pw, pb, ng, nb, wa=None, wb=None):
    with_ab = wa is not None
    full = lambda shape: pl.BlockSpec(shape, lambda i: (0,) * len(shape))
    in_specs = [
        pl.BlockSpec((NC, BN, H), lambda i: (0, i, 0)),
        pl.BlockSpec((NC, BN, H), lambda i: (0, i, 0)),
        pl.BlockSpec((BN, H), lambda i: (i, 0)),
        full((H, H)), full((1, H)), full((1, H)), full((1, H)),
    ]
    out_specs = [pl.BlockSpec((BN, H), lambda i: (i, 0))]
    out_shape = [jax.ShapeDtypeStruct((N, H), jnp.float32)]
    args = [aggp, cntp, h, pw, pb, ng, nb]
    if with_ab:
        in_specs += [full((H, H2)), full((H, H2))]
        out_specs += [pl.BlockSpec((BN, H2), lambda i: (i, 0)),
                      pl.BlockSpec((BN, H2), lambda i: (i, 0))]
        out_shape += [jax.ShapeDtypeStruct((NP, H2), jnp.float32),
                      jax.ShapeDtypeStruct((NP, H2), jnp.float32)]
        args += [wa, wb]
    out = pl.pallas_call(
        _make_post_body(with_ab),
        grid=(N // BN,),
        in_specs=in_specs,
        out_specs=out_specs,
        out_shape=out_shape,
    )(*args)
    return out


def kernel(x, pos, edge_attr, params, edge_index, batch):
    feats = jnp.concatenate([x, pos], axis=1)
    src = edge_index[0]
    dst = edge_index[1]
    pad = N + (jnp.arange(EP - E, dtype=jnp.int32) % (NP - N))
    dsti = jnp.concatenate([dst, pad]).reshape(NCHUNKP, 1, CHUNK)
    srci = jnp.concatenate([src, pad]).reshape(NCHUNKP, 1, CHUNK)
    eap = jnp.pad(edge_attr, ((0, EP - E), (0, 0)))

    row = lambda v: v.reshape(1, -1)
    wa, wb, cmat, cbias = [], [], [], []
    for lp in params['layers']:
        w1 = lp['W1']
        w1a, w1b, w1c = w1[:H], w1[H:2 * H], w1[2 * H:]
        wa.append(w1a - w1b)
        wb.append(w1b)
        cmat.append(params['edge_W'] @ w1c)
        cbias.append(row(params['edge_b'] @ w1c + lp['b1']))

    h, a, b = _node_pre(feats, params['node_W'], row(params['node_b']),
                        wa[0], wb[0])

    z128 = jnp.zeros((NP, H), jnp.float32)
    cntp = _sc_count(dsti, z128)

    for li, lp in enumerate(params['layers']):
        g = _sc_gather(a, b, dsti, srci)
        m2 = _edge_mlp(g, eap, cmat[li], cbias[li], lp['W2'],
                       row(lp['b2']), row(lp['g1']), row(lp['be1']),
                       row(lp['g2']), row(lp['be2']))
        aggp = _sc_scatter(m2, dsti, z128)
        if li == 0:
            h, a, b = _post(aggp, cntp, h, lp['pW'], row(lp['pb']),
                            row(lp['ng']), row(lp['nb']),
                            wa[1], wb[1])
        else:
            h = _post(aggp, cntp, h, lp['pW'], row(lp['pb']),
                      row(lp['ng']), row(lp['nb']))[0]
    return h


# speedup vs baseline: 1.5707x; 1.0292x over previous
import functools
import jax
import jax.numpy as jnp
from jax import lax
from jax.experimental import pallas as pl
from jax.experimental.pallas import tpu as pltpu, tpu_sc as plsc

N = 10000
NP = 10240
E = 160000
H = 128
H2 = 2 * H

NC = 2
NS = 16
NW = NC * NS
CHUNK = 120
NCHUNKP = 1344
EP = NCHUNKP * CHUNK
CPW = NCHUNKP // NW
ROWS_PER_SUB = NP // NS

BN = 1000
BE = 2016

_mesh = plsc.VectorSubcoreMesh(core_axis_name="c", subcore_axis_name="s")


def _sc_gather_body(a_hbm, b_hbm, dsti, srci, g_hbm,
                    idx_d0, idx_s0, idx_d1, idx_s1,
                    ba0, bb0, ba1, bb1, sem_a, sem_b, sem_w):
    c = lax.axis_index("c")
    s = lax.axis_index("s")
    wid = s * NC + c
    idx_d = [idx_d0, idx_d1]
    idx_s = [idx_s0, idx_s1]
    bas = [ba0, ba1]
    bbs = [bb0, bb1]

    def chunk_of(j):
        return j * NW + wid

    pltpu.sync_copy(dsti.at[chunk_of(0), 0], idx_d0)
    pltpu.sync_copy(srci.at[chunk_of(0), 0], idx_s0)
    pltpu.make_async_copy(a_hbm.at[idx_d0], ba0, sem_a).start()
    pltpu.make_async_copy(b_hbm.at[idx_s0], bb0, sem_b).start()
    pltpu.sync_copy(dsti.at[chunk_of(1), 0], idx_d1)
    pltpu.sync_copy(srci.at[chunk_of(1), 0], idx_s1)

    def pair_body(jj, carry):
        for bpar in range(2):
            j = jj * 2 + bpar
            cur, nxt = bpar, 1 - bpar

            pltpu.make_async_copy(a_hbm.at[idx_d[cur]], bas[cur], sem_a).wait()
            pltpu.make_async_copy(b_hbm.at[idx_s[cur]], bbs[cur], sem_b).wait()

            @pl.when(j >= 1)
            def _():
                pltpu.make_async_copy(
                    bas[nxt], g_hbm.at[pl.ds(0, CHUNK)], sem_w).wait()

            @pl.when(j + 1 < CPW)
            def _():
                pltpu.make_async_copy(
                    a_hbm.at[idx_d[nxt]], bas[nxt], sem_a).start()
                pltpu.make_async_copy(
                    b_hbm.at[idx_s[nxt]], bbs[nxt], sem_b).start()

            @pl.when(j + 2 < CPW)
            def _():
                pltpu.sync_copy(dsti.at[chunk_of(j + 2), 0], idx_d[cur])
                pltpu.sync_copy(srci.at[chunk_of(j + 2), 0], idx_s[cur])

            @plsc.parallel_loop(0, CHUNK, 1, unroll=4)
            def _row(i):
                for g in range(H2 // 16):
                    sl = pl.ds(g * 16, 16)
                    bas[cur][i, sl] = bas[cur][i, sl] + bbs[cur][i, sl]

            pltpu.make_async_copy(
                bas[cur], g_hbm.at[pl.ds(chunk_of(j) * CHUNK, CHUNK)],
                sem_w).start()
        return carry

    lax.fori_loop(0, CPW // 2, pair_body, 0)
    pltpu.make_async_copy(bas[1], g_hbm.at[pl.ds(0, CHUNK)], sem_w).wait()


_sc_gather = pl.kernel(
    _sc_gather_body,
    out_type=jax.ShapeDtypeStruct((EP, H2), jnp.float32),
    mesh=_mesh,
    scratch_types=[
        pltpu.VMEM((CHUNK,), jnp.int32),
        pltpu.VMEM((CHUNK,), jnp.int32),
        pltpu.VMEM((CHUNK,), jnp.int32),
        pltpu.VMEM((CHUNK,), jnp.int32),
        pltpu.VMEM((CHUNK, H2), jnp.float32),
        pltpu.VMEM((CHUNK, H2), jnp.float32),
        pltpu.VMEM((CHUNK, H2), jnp.float32),
        pltpu.VMEM((CHUNK, H2), jnp.float32),
        pltpu.SemaphoreType.DMA,
        pltpu.SemaphoreType.DMA,
        pltpu.SemaphoreType.DMA,
    ],
)


def _sc_count_body(dsti, z128, cnt_out, idx, ones_v, cnt_sh):
    c = lax.axis_index("c")
    s = lax.axis_index("s")
    wid = s * NC + c

    @pl.when(s == 0)
    def _zero():
        pltpu.sync_copy(z128, cnt_sh)

    def ones_body(i, carry):
        for g in range(H // 16):
            ones_v[i, pl.ds(g * 16, 16)] = jnp.full((16,), 1.0, jnp.float32)
        return carry
    lax.fori_loop(0, CHUNK, ones_body, 0)

    plsc.subcore_barrier()

    def chunk_body(j, carry):
        ch = j * NW + wid
        pltpu.sync_copy(dsti.at[ch, 0], idx)
        pltpu.sync_copy(ones_v, cnt_sh.at[idx], add=True)
        return carry

    lax.fori_loop(0, CPW, chunk_body, 0)
    plsc.subcore_barrier()

    rows = pl.ds(s * ROWS_PER_SUB, ROWS_PER_SUB)
    pltpu.sync_copy(cnt_sh.at[rows], cnt_out.at[c, rows])


_sc_count = pl.kernel(
    _sc_count_body,
    out_type=jax.ShapeDtypeStruct((NC, NP, H), jnp.float32),
    mesh=_mesh,
    scratch_types=[
        pltpu.VMEM((CHUNK,), jnp.int32),
        pltpu.VMEM((CHUNK, H), jnp.float32),
        pltpu.VMEM_SHARED((NP, H), jnp.float32),
    ],
)


def _sc_scatter_body(m2_hbm, dsti, z128, agg_out,
                     idx0, idx1, mb0, mb1, agg_sh, sem_m):
    c = lax.axis_index("c")
    s = lax.axis_index("s")
    wid = s * NC + c
    idxs = [idx0, idx1]
    mbs = [mb0, mb1]

    @pl.when(s == 0)
    def _zero():
        pltpu.sync_copy(z128, agg_sh)

    def chunk_of(j):
        return j * NW + wid

    pltpu.sync_copy(dsti.at[chunk_of(0), 0], idx0)
    pltpu.make_async_copy(
        m2_hbm.at[pl.ds(chunk_of(0) * CHUNK, CHUNK)], mb0, sem_m).start()
    pltpu.sync_copy(dsti.at[chunk_of(1), 0], idx1)

    plsc.subcore_barrier()

    def pair_body(jj, carry):
        for bpar in range(2):
            j = jj * 2 + bpar
            cur, nxt = bpar, 1 - bpar

            pltpu.make_async_copy(
                m2_hbm.at[pl.ds(0, CHUNK)], mbs[cur], sem_m).wait()

            @pl.when(j + 1 < CPW)
            def _():
                pltpu.make_async_copy(
                    m2_hbm.at[pl.ds(chunk_of(j + 1) * CHUNK, CHUNK)],
                    mbs[nxt], sem_m).start()

            pltpu.sync_copy(mbs[cur], agg_sh.at[idxs[cur]], add=True)

            @pl.when(j + 2 < CPW)
            def _():
                pltpu.sync_copy(dsti.at[chunk_of(j + 2), 0], idxs[cur])
        return carry

    lax.fori_loop(0, CPW // 2, pair_body, 0)
    plsc.subcore_barrier()

    rows = pl.ds(s * ROWS_PER_SUB, ROWS_PER_SUB)
    pltpu.sync_copy(agg_sh.at[rows], agg_out.at[c, rows])


_sc_scatter = pl.kernel(
    _sc_scatter_body,
    out_type=jax.ShapeDtypeStruct((NC, NP, H), jnp.float32),
    mesh=_mesh,
    scratch_types=[
        pltpu.VMEM((CHUNK,), jnp.int32),
        pltpu.VMEM((CHUNK,), jnp.int32),
        pltpu.VMEM((CHUNK, H), jnp.float32),
        pltpu.VMEM((CHUNK, H), jnp.float32),
        pltpu.VMEM_SHARED((NP, H), jnp.float32),
        pltpu.SemaphoreType.DMA,
    ],
)


def _ln_relu(z, g, b):
    mu = jnp.mean(z, axis=-1, keepdims=True)
    zc = z - mu
    var = jnp.mean(zc * zc, axis=-1, keepdims=True)
    return jax.nn.relu(zc * jax.lax.rsqrt(var + 1e-5) * g + b)


def _dot(a, b):
    return jnp.dot(a, b, preferred_element_type=jnp.float32)


def _node_pre_body(x_ref, pos_ref, nw_ref, nb_ref, wa_ref, wb_ref,
                   h_ref, a_ref, b_ref):
    feats = jnp.concatenate([x_ref[...], pos_ref[...]], axis=1)
    h = _dot(feats, nw_ref[...]) + nb_ref[...]
    h_ref[...] = h
    a_ref[...] = _dot(h, wa_ref[...])
    b_ref[...] = _dot(h, wb_ref[...])


def _node_pre(x, pos, nw, nb, wa, wb):
    full = lambda shape: pl.BlockSpec(shape, lambda i: (0,) * len(shape))
    return pl.pallas_call(
        _node_pre_body,
        grid=(N // BN,),
        in_specs=[
            pl.BlockSpec((BN, 125), lambda i: (i, 0)),
            pl.BlockSpec((BN, 3), lambda i: (i, 0)),
            full((H, H)), full((1, H)), full((H, H2)), full((H, H2)),
        ],
        out_specs=[
            pl.BlockSpec((BN, H), lambda i: (i, 0)),
            pl.BlockSpec((BN, H2), lambda i: (i, 0)),
            pl.BlockSpec((BN, H2), lambda i: (i, 0)),
        ],
        out_shape=[
            jax.ShapeDtypeStruct((N, H), jnp.float32),
            jax.ShapeDtypeStruct((NP, H2), jnp.float32),
            jax.ShapeDtypeStruct((NP, H2), jnp.float32),
        ],
    )(x, pos, nw, nb, wa, wb)


def _edge_mlp_body(g_ref, ea_ref, c_ref, cb_ref, w2_ref, b2_ref,
                   g1_ref, be1_ref, g2_ref, be2_ref, out_ref):
    z = g_ref[...] + _dot(ea_ref[...], c_ref[...]) + cb_ref[...]
    m = _ln_relu(z, g1_ref[...], be1_ref[...])
    m2 = _dot(m, w2_ref[...]) + b2_ref[...]
    out_ref[...] = _ln_relu(m2, g2_ref[...], be2_ref[...])


def _edge_mlp(g, ea, cmat, cbias, w2, b2, g1, be1, g2, be2):
    full = lambda shape: pl.BlockSpec(shape, lambda i: (0,) * len(shape))
    return pl.pallas_call(
        _edge_mlp_body,
        grid=(EP // BE,),
        in_specs=[
            pl.BlockSpec((BE, H2), lambda i: (i, 0)),
            pl.BlockSpec((BE, 16), lambda i: (i, 0)),
            full((16, H2)), full((1, H2)), full((H2, H)), full((1, H)),
            full((1, H2)), full((1, H2)), full((1, H)), full((1, H)),
        ],
        out_specs=pl.BlockSpec((BE, H), lambda i: (i, 0)),
        out_shape=jax.ShapeDtypeStruct((EP, H), jnp.float32),
    )(g, ea, cmat, cbias, w2, b2, g1, be1, g2, be2)


def _make_post_body(with_ab):
    def body(*refs):
        if with_ab:
            (aggp_ref, cntp_ref, h_ref, pw_ref, pb_ref, ng_ref, nb_ref,
             wa_ref, wb_ref, out_ref, a_ref, b_ref) = refs
        else:
            (aggp_ref, cntp_ref, h_ref, pw_ref, pb_ref, ng_ref, nb_ref,
             out_ref) = refs
        agg = aggp_ref[0] + aggp_ref[1]
        cnt = cntp_ref[0, :, 0] + cntp_ref[1, :, 0]
        agg = agg / jnp.maximum(cnt, 1.0)[:, None]
        o = _dot(agg, pw_ref[...]) + pb_ref[...]
        hn = _ln_relu(o, ng_ref[...], nb_ref[...]) + h_ref[...]
        out_ref[...] = hn
        if with_ab:
            a_ref[...] = _dot(hn, wa_ref[...])
            b_ref[...] = _dot(hn, wb_ref[...])
    return body


def _post(aggp, cntp, h, pw, pb, ng, nb, wa=None, wb=None):
    with_ab = wa is not None
    full = lambda shape: pl.BlockSpec(shape, lambda i: (0,) * len(shape))
    in_specs = [
        pl.BlockSpec((NC, BN, H), lambda i: (0, i, 0)),
        pl.BlockSpec((NC, BN, H), lambda i: (0, i, 0)),
        pl.BlockSpec((BN, H), lambda i: (i, 0)),
        full((H, H)), full((1, H)), full((1, H)), full((1, H)),
    ]
    out_specs = [pl.BlockSpec((BN, H), lambda i: (i, 0))]
    out_shape = [jax.ShapeDtypeStruct((N, H), jnp.float32)]
    args = [aggp, cntp, h, pw, pb, ng, nb]
    if with_ab:
        in_specs += [full((H, H2)), full((H, H2))]
        out_specs += [pl.BlockSpec((BN, H2), lambda i: (i, 0)),
                      pl.BlockSpec((BN, H2), lambda i: (i, 0))]
        out_shape += [jax.ShapeDtypeStruct((NP, H2), jnp.float32),
                      jax.ShapeDtypeStruct((NP, H2), jnp.float32)]
        args += [wa, wb]
    out = pl.pallas_call(
        _make_post_body(with_ab),
        grid=(N // BN,),
        in_specs=in_specs,
        out_specs=out_specs,
        out_shape=out_shape,
    )(*args)
    return out


def kernel(x, pos, edge_attr, params, edge_index, batch):
    src = edge_index[0]
    dst = edge_index[1]
    pad = N + (jnp.arange(EP - E, dtype=jnp.int32) % (NP - N))
    dsti = jnp.concatenate([dst, pad]).reshape(NCHUNKP, 1, CHUNK)
    srci = jnp.concatenate([src, pad]).reshape(NCHUNKP, 1, CHUNK)

    row = lambda v: v.reshape(1, -1)
    wa, wb, cmat, cbias = [], [], [], []
    for lp in params['layers']:
        w1 = lp['W1']
        w1a, w1b, w1c = w1[:H], w1[H:2 * H], w1[2 * H:]
        wa.append(w1a - w1b)
        wb.append(w1b)
        cmat.append(params['edge_W'] @ w1c)
        cbias.append(row(params['edge_b'] @ w1c + lp['b1']))

    h, a, b = _node_pre(x, pos, params['node_W'], row(params['node_b']),
                        wa[0], wb[0])

    z128 = jnp.zeros((NP, H), jnp.float32)
    cntp = _sc_count(dsti, z128)

    for li, lp in enumerate(params['layers']):
        g = _sc_gather(a, b, dsti, srci)
        m2 = _edge_mlp(g, edge_attr, cmat[li], cbias[li], lp['W2'],
                       row(lp['b2']), row(lp['g1']), row(lp['be1']),
                       row(lp['g2']), row(lp['be2']))
        aggp = _sc_scatter(m2, dsti, z128)
        if li == 0:
            h, a, b = _post(aggp, cntp, h, lp['pW'], row(lp['pb']),
                            row(lp['ng']), row(lp['nb']),
                            wa[1], wb[1])
        else:
            h = _post(aggp, cntp, h, lp['pW'], row(lp['pb']),
                      row(lp['ng']), row(lp['nb']))[0]
    return h
